# baseline (speedup 1.0000x reference)
import functools
import jax
import jax.numpy as jnp
from jax import lax
from jax.experimental import pallas as pl
from jax.experimental.pallas import tpu as pltpu
from jax.experimental.pallas import tpu_sc as plsc

_D = 256
_I = 128
_J = 128
_GRP = 16


def _sc_body(s, n_emb, emb_hbm, out_hbm, tbl_r, idx_v, sem):
    n_rows = _I + _J
    wid = lax.axis_index("s")
    it = wid // (s // _J)
    jt = wid % (s // _J)
    i0 = it * _I
    j0 = jt * _J
    r0 = i0 - j0 + (s - 1) - (_J - 1)

    lane = lax.broadcasted_iota(jnp.int32, (16,), 0)
    for h in range(2):
        for c in range(8):
            a = r0 + (n_rows - 1) - 128 * h - 16 * c
            idx_v[h, pl.ds(c * 16, 16)] = a - lane

    for h in range(2):
        pltpu.make_async_copy(
            emb_hbm.at[idx_v.at[h]], tbl_r.at[pl.ds(128 * h, 128)], sem).start()
    for h in range(2):
        pltpu.make_async_copy(
            emb_hbm.at[idx_v.at[h]], tbl_r.at[pl.ds(128 * h, 128)], sem).wait()

    def copy_desc(li):
        return pltpu.make_async_copy(
            tbl_r.at[pl.ds(_I - li, _J)],
            out_hbm.at[pl.ds((i0 + li) * s + j0, _J)],
            sem,
        )

    def fire_group(g, _):
        for b in range(_GRP):
            copy_desc(g * _GRP + b).start()
        return _

    def drain_group(g, _):
        for b in range(_GRP):
            copy_desc(g * _GRP + b).wait()
        return _

    lax.fori_loop(0, _I // _GRP, fire_group, None)
    lax.fori_loop(0, _I // _GRP, drain_group, None)


def kernel(seq_len, rel_pos_emb):
    n_emb, d = rel_pos_emb.shape
    s = (n_emb + 1) // 2

    mesh = plsc.VectorSubcoreMesh(
        core_axis_name="c", subcore_axis_name="s", num_cores=1)
    body = functools.partial(_sc_body, s, n_emb)
    sc_kernel = pl.kernel(
        body,
        mesh=mesh,
        out_type=jax.ShapeDtypeStruct((s * s, d), rel_pos_emb.dtype),
        scratch_types=[
            pltpu.VMEM((_I + _J, d), rel_pos_emb.dtype),
            pltpu.VMEM((2, 128), jnp.int32),
            pltpu.SemaphoreType.DMA,
        ],
        compiler_params=pltpu.CompilerParams(use_tc_tiling_on_sc=False),
    )
    emb_pad = jnp.concatenate(
        [rel_pos_emb, jnp.zeros((1, d), rel_pos_emb.dtype)], axis=0)
    out = sc_kernel(emb_pad)
    return out.reshape(s, s, d)


# speedup vs baseline: 4.3392x; 4.3392x over previous
import functools
import jax
import jax.numpy as jnp
from jax import lax
from jax.experimental import pallas as pl
from jax.experimental.pallas import tpu as pltpu
from jax.experimental.pallas import tpu_sc as plsc

_NBUF = 16
_CHUNK = 64


def _sc_reverse_body(n_pad, emb_hbm, femb_hbm, idx_v, buf_v, sem):
    wid = lax.axis_index("s")
    k0 = wid * _CHUNK
    lane = lax.broadcasted_iota(jnp.int32, (16,), 0)
    for c in range(_CHUNK // 16):
        v = (n_pad - 2 - k0 - 16 * c) - lane
        idx_v[pl.ds(c * 16, 16)] = jnp.where(v < 0, n_pad - 1, v)
    gather = pltpu.make_async_copy(emb_hbm.at[idx_v], buf_v, sem)
    gather.start()
    gather.wait()
    pltpu.sync_copy(buf_v, femb_hbm.at[pl.ds(k0, _CHUNK)])


def _tc_stream_body(s, n_emb, n_pad, base_ref, femb_ref, out_ref,
                    femb8_ref, sems):
    i = pl.program_id(0)

    @pl.when(i == 0)
    def _():
        femb = femb_ref[...]
        for p in range(8):
            femb8_ref[p] = pltpu.roll(femb, (n_pad - p) % n_pad, 0)

    start = base_ref[0] - i
    p = jax.lax.rem(start, 8)
    a = pl.multiple_of(start - p, 8)

    @pl.when(i >= _NBUF)
    def _():
        pltpu.make_async_copy(
            femb8_ref.at[0, pl.ds(0, s), :], out_ref.at[0], sems.at[i % _NBUF]
        ).wait()

    pltpu.make_async_copy(
        femb8_ref.at[p, pl.ds(a, s), :], out_ref.at[i], sems.at[i % _NBUF]
    ).start()

    @pl.when(i == s - 1)
    def _():
        for k in range(_NBUF):
            pltpu.make_async_copy(
                femb8_ref.at[0, pl.ds(0, s), :], out_ref.at[0], sems.at[k]
            ).wait()


def kernel(seq_len, rel_pos_emb):
    n_emb, d = rel_pos_emb.shape
    s = (n_emb + 1) // 2
    n_pad = n_emb + 1
    base = n_emb - seq_len

    mesh = plsc.VectorSubcoreMesh(
        core_axis_name="c", subcore_axis_name="s", num_cores=1)
    sc_reverse = pl.kernel(
        functools.partial(_sc_reverse_body, n_pad),
        mesh=mesh,
        out_type=jax.ShapeDtypeStruct((n_pad, d), rel_pos_emb.dtype),
        scratch_types=[
            pltpu.VMEM((_CHUNK,), jnp.int32),
            pltpu.VMEM((_CHUNK, d), rel_pos_emb.dtype),
            pltpu.SemaphoreType.DMA,
        ],
        compiler_params=pltpu.CompilerParams(use_tc_tiling_on_sc=False),
    )
    emb_pad = jnp.concatenate(
        [rel_pos_emb, jnp.zeros((1, d), rel_pos_emb.dtype)], axis=0)
    femb = sc_reverse(emb_pad)

    out = pl.pallas_call(
        functools.partial(_tc_stream_body, s, n_emb, n_pad),
        grid_spec=pltpu.PrefetchScalarGridSpec(
            num_scalar_prefetch=1,
            grid=(s,),
            in_specs=[pl.BlockSpec((n_pad, d), lambda i, base: (0, 0))],
            out_specs=pl.BlockSpec(memory_space=pl.ANY),
            scratch_shapes=[
                pltpu.VMEM((8, n_pad, d), rel_pos_emb.dtype),
                pltpu.SemaphoreType.DMA((_NBUF,)),
            ],
        ),
        out_shape=jax.ShapeDtypeStruct((s, s, d), rel_pos_emb.dtype),
    )(jnp.asarray(base, jnp.int32).reshape(1), femb)
    return out
